# initial kernel scaffold (unmeasured)
import jax
import jax.numpy as jnp
from jax import lax
from jax.experimental import pallas as pl
from jax.experimental.pallas import tpu as pltpu


def kernel(
    x,
):
    def body(*refs):
        pass

    out_shape = jax.ShapeDtypeStruct(..., jnp.float32)
    return pl.pallas_call(body, out_shape=out_shape)(...)


# baseline (device time: 7073 ns/iter reference)
import jax
import jax.numpy as jnp
from jax import lax
from jax.experimental import pallas as pl
from jax.experimental.pallas import tpu as pltpu

N_DEV = 4


def kernel(x):
    m, n = x.shape

    def body(x_ref, out_ref, halo_ref, send_sems, recv_sems):
        my_i = lax.axis_index("i")
        left = (my_i - 1) % N_DEV
        right = (my_i + 1) % N_DEV

        barrier_sem = pltpu.get_barrier_semaphore()
        for nbr in (left, right):
            pl.semaphore_signal(
                barrier_sem, inc=1,
                device_id=(nbr,), device_id_type=pl.DeviceIdType.MESH,
            )
        pl.semaphore_wait(barrier_sem, 2)

        send_right = pltpu.make_async_remote_copy(
            src_ref=x_ref.at[pl.ds(m - 1, 1), :],
            dst_ref=halo_ref.at[0],
            send_sem=send_sems.at[0],
            recv_sem=recv_sems.at[0],
            device_id=(right,),
            device_id_type=pl.DeviceIdType.MESH,
        )
        send_left = pltpu.make_async_remote_copy(
            src_ref=x_ref.at[pl.ds(0, 1), :],
            dst_ref=halo_ref.at[1],
            send_sem=send_sems.at[1],
            recv_sem=recv_sems.at[1],
            device_id=(left,),
            device_id_type=pl.DeviceIdType.MESH,
        )
        send_right.start()
        send_left.start()

        out_ref[pl.ds(1, m - 2), :] = (
            0.25 * x_ref[pl.ds(0, m - 2), :]
            + 0.5 * x_ref[pl.ds(1, m - 2), :]
            + 0.25 * x_ref[pl.ds(2, m - 2), :]
        )

        send_right.wait()
        send_left.wait()

        top = 0.25 * halo_ref[0, 0, :] + 0.5 * x_ref[0, :] + 0.25 * x_ref[1, :]
        out_ref[0, :] = jnp.where(my_i == 0, x_ref[0, :], top)

        bot = 0.25 * x_ref[m - 2, :] + 0.5 * x_ref[m - 1, :] + 0.25 * halo_ref[1, 0, :]
        out_ref[m - 1, :] = jnp.where(my_i == N_DEV - 1, x_ref[m - 1, :], bot)

    return pl.pallas_call(
        body,
        out_shape=jax.ShapeDtypeStruct((m, n), x.dtype),
        in_specs=[pl.BlockSpec(memory_space=pltpu.VMEM)],
        out_specs=pl.BlockSpec(memory_space=pltpu.VMEM),
        scratch_shapes=[
            pltpu.VMEM((2, 1, n), x.dtype),
            pltpu.SemaphoreType.DMA((2,)),
            pltpu.SemaphoreType.DMA((2,)),
        ],
        compiler_params=pltpu.CompilerParams(collective_id=0),
    )(x)


# device time: 5492 ns/iter; 1.2879x vs baseline; 1.2879x over previous
import jax
import jax.numpy as jnp
from jax import lax
from jax.experimental import pallas as pl
from jax.experimental.pallas import tpu as pltpu

N_DEV = 4


def kernel(x):
    m, n = x.shape
    x = pltpu.with_memory_space_constraint(x, pltpu.MemorySpace.HBM)

    def body(x_any, out_ref, x_vmem, send_buf, halo_ref,
             local_sems, send_sems, recv_sems):
        my_i = lax.axis_index("i")
        left = (my_i - 1) % N_DEV
        right = (my_i + 1) % N_DEV
        has_left = my_i > 0
        has_right = my_i < N_DEV - 1

        barrier_sem = pltpu.get_barrier_semaphore()

        @pl.when(has_left)
        def _():
            pl.semaphore_signal(barrier_sem, inc=1, device_id=(left,),
                                device_id_type=pl.DeviceIdType.MESH)

        @pl.when(has_right)
        def _():
            pl.semaphore_signal(barrier_sem, inc=1, device_id=(right,),
                                device_id_type=pl.DeviceIdType.MESH)

        stage_top = pltpu.make_async_copy(
            x_any.at[pl.ds(0, 8), :], send_buf.at[0], local_sems.at[0])
        stage_bot = pltpu.make_async_copy(
            x_any.at[pl.ds(m - 8, 8), :], send_buf.at[1], local_sems.at[1])
        stage_top.start()
        stage_bot.start()
        cp_in = pltpu.make_async_copy(x_any, x_vmem, local_sems.at[2])
        cp_in.start()

        send_right = pltpu.make_async_remote_copy(
            src_ref=send_buf.at[1].at[pl.ds(7, 1), :],
            dst_ref=halo_ref.at[0],
            send_sem=send_sems.at[0],
            recv_sem=recv_sems.at[0],
            device_id=(right,),
            device_id_type=pl.DeviceIdType.MESH,
        )
        send_left = pltpu.make_async_remote_copy(
            src_ref=send_buf.at[0].at[pl.ds(0, 1), :],
            dst_ref=halo_ref.at[1],
            send_sem=send_sems.at[1],
            recv_sem=recv_sems.at[1],
            device_id=(left,),
            device_id_type=pl.DeviceIdType.MESH,
        )
        @pl.when(has_left & has_right)
        def _():
            pl.semaphore_wait(barrier_sem, 2)

        @pl.when(jnp.logical_not(has_left & has_right))
        def _():
            pl.semaphore_wait(barrier_sem, 1)

        stage_top.wait()

        @pl.when(has_left)
        def _():
            send_left.start()

        stage_bot.wait()

        @pl.when(has_right)
        def _():
            send_right.start()

        cp_in.wait()
        out_ref[pl.ds(1, m - 2), :] = (
            0.25 * x_vmem[pl.ds(0, m - 2), :]
            + 0.5 * x_vmem[pl.ds(1, m - 2), :]
            + 0.25 * x_vmem[pl.ds(2, m - 2), :]
        )

        @pl.when(has_left)
        def _():
            send_right.wait_recv()

        top = (0.25 * halo_ref[0, 0, :] + 0.5 * x_vmem[0, :]
               + 0.25 * x_vmem[1, :])
        out_ref[0, :] = jnp.where(my_i == 0, x_vmem[0, :], top)

        @pl.when(has_right)
        def _():
            send_left.wait_recv()

        bot = (0.25 * x_vmem[m - 2, :] + 0.5 * x_vmem[m - 1, :]
               + 0.25 * halo_ref[1, 0, :])
        out_ref[m - 1, :] = jnp.where(my_i == N_DEV - 1, x_vmem[m - 1, :], bot)

        @pl.when(has_right)
        def _():
            send_right.wait_send()

        @pl.when(has_left)
        def _():
            send_left.wait_send()

    return pl.pallas_call(
        body,
        out_shape=jax.ShapeDtypeStruct((m, n), x.dtype),
        in_specs=[pl.BlockSpec(memory_space=pltpu.MemorySpace.HBM)],
        out_specs=pl.BlockSpec(memory_space=pltpu.VMEM),
        scratch_shapes=[
            pltpu.VMEM((m, n), x.dtype),
            pltpu.VMEM((2, 8, n), x.dtype),
            pltpu.VMEM((2, 1, n), x.dtype),
            pltpu.SemaphoreType.DMA((3,)),
            pltpu.SemaphoreType.DMA((2,)),
            pltpu.SemaphoreType.DMA((2,)),
        ],
        compiler_params=pltpu.CompilerParams(collective_id=0),
    )(x)
